# baseline (speedup 1.0000x reference)
import functools

import jax
import jax.numpy as jnp
from jax import lax
from jax.experimental import pallas as pl
from jax.experimental.pallas import tpu as pltpu
from jax.experimental.pallas import tpu_sc as plsc

VOCAB = 1_000_000
DIM = 32
LANE = 128
NW = 32
N_VBLK = VOCAB // LANE
V_TAIL = VOCAB - N_VBLK * LANE
A_STEPS = (N_VBLK + NW - 1) // NW

NI_TILE = 16384 // LANE
N_FULL_UNIT = 6 * NI_TILE
B_FULL_STEPS = N_FULL_UNIT // NW
B_TAIL_STEPS = NI_TILE // NW

_mesh = plsc.VectorSubcoreMesh(core_axis_name="c", subcore_axis_name="s")
_params = pltpu.CompilerParams(needs_layout_passes=False)


def _iota16():
    return lax.iota(jnp.int32, 16)


@functools.partial(
    pl.kernel,
    mesh=_mesh,
    out_type=jax.ShapeDtypeStruct((VOCAB // 4, LANE), jnp.float32),
    scratch_types=[
        pltpu.VMEM((DIM, LANE), jnp.float32),
        pltpu.VMEM((DIM, LANE), jnp.float32),
    ],
    compiler_params=_params,
)
def _transpose_table(tabT_hbm, tail16_hbm, t4_hbm, in_v, out_v):
    wid = lax.axis_index("s") * 2 + lax.axis_index("c")

    def transpose_nq(nq):
        def q_body(q, carry):
            for c in range(8):
                r = q * 4 + c // 2
                f0 = (c % 2) * 16
                vals = plsc.load_gather(
                    in_v, [f0 + _iota16(), jnp.full((16,), r, jnp.int32)])
                out_v[q, pl.ds(c * 16, 16)] = vals
            return carry

        lax.fori_loop(0, nq, q_body, 0)

    def step(k, carry):
        b = wid + k * NW

        @pl.when(b < N_VBLK)
        def _():
            pltpu.sync_copy(tabT_hbm.at[:, pl.ds(b * LANE, LANE)], in_v)
            transpose_nq(32)
            pltpu.sync_copy(out_v, t4_hbm.at[pl.ds(b * 32, 32)])

        return carry

    lax.fori_loop(0, A_STEPS, step, 0)

    @pl.when(wid == 4)
    def _():
        pltpu.sync_copy(tail16_hbm, in_v.at[pl.ds(0, 16)])
        pltpu.sync_copy(
            in_v.at[pl.ds(0, 16)], t4_hbm.at[pl.ds(N_VBLK * 32, 16)])


@functools.partial(
    pl.kernel,
    mesh=_mesh,
    out_type=jax.ShapeDtypeStruct((50, DIM, 16384), jnp.float32),
    scratch_types=[
        pltpu.VMEM((8, LANE), jnp.int32),
        pltpu.VMEM((LANE,), jnp.int32),
        pltpu.VMEM((LANE,), jnp.int32),
        pltpu.VMEM((LANE, LANE), jnp.float32),
        pltpu.VMEM((DIM, LANE), jnp.float32),
        pltpu.SemaphoreType.DMA,
    ],
    compiler_params=_params,
)
def _gather_out(t4_hbm, xT_hbm, out_hbm, idx_v, tl_v, ml_v, buf_v, ob_v, sem):
    wid = lax.axis_index("s") * 2 + lax.axis_index("c")

    def do_j(jg, jl, it):
        for ci in range(8):
            v = idx_v[jl, pl.ds(ci * 16, 16)]
            tl_v[pl.ds(ci * 16, 16)] = lax.shift_right_logical(v, 2)
            ml_v[pl.ds(ci * 16, 16)] = (v & 3) * DIM
        pltpu.async_copy(t4_hbm.at[tl_v], buf_v, sem).wait()

        for f in range(DIM):
            def ic_body(ic, c3, f=f):
                m = ml_v[pl.ds(ic * 16, 16)]
                vals = plsc.load_gather(
                    buf_v, [ic * 16 + _iota16(), m + f])
                ob_v[f, pl.ds(ic * 16, 16)] = vals
                return c3

            lax.fori_loop(0, 8, ic_body, 0)

        pltpu.sync_copy(ob_v, out_hbm.at[jg, :, pl.ds(it * LANE, LANE)])

    def full_step(k, carry):
        u = wid + k * NW
        jb = u // NI_TILE
        it = u - jb * NI_TILE
        pltpu.sync_copy(
            xT_hbm.at[pl.ds(jb * 8, 8), pl.ds(it * LANE, LANE)], idx_v)
        for jl in range(8):
            do_j(jb * 8 + jl, jl, it)
        return carry

    def tail_step(k, carry):
        it = wid + k * NW
        pltpu.sync_copy(
            xT_hbm.at[pl.ds(48, 2), pl.ds(it * LANE, LANE)],
            idx_v.at[pl.ds(0, 2)])
        for jl in range(2):
            do_j(48 + jl, jl, it)
        return carry

    lax.fori_loop(0, B_FULL_STEPS, full_step, 0)
    lax.fori_loop(0, B_TAIL_STEPS, tail_step, 0)


def kernel(x, table):
    tabT = table.T
    xT = x.astype(jnp.int32).T
    tail16 = table[N_VBLK * LANE:, :].reshape(16, LANE)
    t4 = _transpose_table(tabT, tail16)
    out_p = _gather_out(t4, xT)
    return jnp.transpose(out_p, (2, 0, 1))


# speedup vs baseline: 1.1994x; 1.1994x over previous
import functools

import jax
import jax.numpy as jnp
from jax import lax
from jax.experimental import pallas as pl
from jax.experimental.pallas import tpu as pltpu
from jax.experimental.pallas import tpu_sc as plsc

VOCAB = 1_000_000
DIM = 32
LANE = 128
NW = 32
N_VBLK = VOCAB // LANE
V_TAIL = VOCAB - N_VBLK * LANE
A_STEPS = (N_VBLK + NW - 1) // NW

NI_TILE = 16384 // LANE
N_FULL_UNIT = 6 * NI_TILE
B_FULL_STEPS = N_FULL_UNIT // NW
B_TAIL_STEPS = NI_TILE // NW

_mesh = plsc.VectorSubcoreMesh(core_axis_name="c", subcore_axis_name="s")
_params = pltpu.CompilerParams(needs_layout_passes=False)


def _iota16():
    return lax.iota(jnp.int32, 16)


@functools.partial(
    pl.kernel,
    mesh=_mesh,
    out_type=jax.ShapeDtypeStruct((VOCAB // 4, LANE), jnp.float32),
    scratch_types=[
        pltpu.VMEM((DIM, LANE), jnp.float32),
        pltpu.VMEM((DIM, LANE), jnp.float32),
    ],
    compiler_params=_params,
)
def _transpose_table(tabT_hbm, tail16_hbm, t4_hbm, in_v, out_v):
    wid = lax.axis_index("s") * 2 + lax.axis_index("c")

    lo16 = _iota16()
    hi16 = 16 + _iota16()

    def transpose_block():
        for q in range(32):
            for c in range(8):
                rows = lo16 if c % 2 == 0 else hi16
                cols = jnp.full((16,), q * 4 + c // 2, jnp.int32)
                out_v[q, pl.ds(c * 16, 16)] = plsc.load_gather(
                    in_v, [rows, cols])

    def step(k, carry):
        b = wid + k * NW

        @pl.when(b < N_VBLK)
        def _():
            pltpu.sync_copy(tabT_hbm.at[:, pl.ds(b * LANE, LANE)], in_v)
            transpose_block()
            pltpu.sync_copy(out_v, t4_hbm.at[pl.ds(b * 32, 32)])

        return carry

    lax.fori_loop(0, A_STEPS, step, 0)

    @pl.when(wid == 4)
    def _():
        pltpu.sync_copy(tail16_hbm, in_v.at[pl.ds(0, 16)])
        pltpu.sync_copy(
            in_v.at[pl.ds(0, 16)], t4_hbm.at[pl.ds(N_VBLK * 32, 16)])


@functools.partial(
    pl.kernel,
    mesh=_mesh,
    out_type=jax.ShapeDtypeStruct((50, DIM, 16384), jnp.float32),
    scratch_types=[
        pltpu.VMEM((8, LANE), jnp.int32),
        pltpu.VMEM((LANE,), jnp.int32),
        pltpu.VMEM((LANE,), jnp.int32),
        pltpu.VMEM((LANE, LANE), jnp.float32),
        pltpu.VMEM((DIM, LANE), jnp.float32),
        pltpu.SemaphoreType.DMA,
    ],
    compiler_params=_params,
)
def _gather_out(t4_hbm, xT_hbm, out_hbm, idx_v, tl_v, ml_v, buf_v, ob_v, sem):
    wid = lax.axis_index("s") * 2 + lax.axis_index("c")

    def do_j(jg, jl, it):
        for ci in range(8):
            v = idx_v[jl, pl.ds(ci * 16, 16)]
            tl_v[pl.ds(ci * 16, 16)] = lax.shift_right_logical(v, 2)
            ml_v[pl.ds(ci * 16, 16)] = (v & 3) * DIM
        pltpu.async_copy(t4_hbm.at[tl_v], buf_v, sem).wait()

        for ic in range(8):
            m = ml_v[pl.ds(ic * 16, 16)]
            rows = ic * 16 + _iota16()
            for f in range(DIM):
                ob_v[f, pl.ds(ic * 16, 16)] = plsc.load_gather(
                    buf_v, [rows, m + f])

        pltpu.sync_copy(ob_v, out_hbm.at[jg, :, pl.ds(it * LANE, LANE)])

    def full_step(k, carry):
        u = wid + k * NW
        jb = u // NI_TILE
        it = u - jb * NI_TILE
        pltpu.sync_copy(
            xT_hbm.at[pl.ds(jb * 8, 8), pl.ds(it * LANE, LANE)], idx_v)

        def jl_body(jl, c2):
            do_j(jb * 8 + jl, jl, it)
            return c2

        lax.fori_loop(0, 8, jl_body, 0)
        return carry

    def tail_step(k, carry):
        it = wid + k * NW
        pltpu.sync_copy(
            xT_hbm.at[pl.ds(48, 2), pl.ds(it * LANE, LANE)],
            idx_v.at[pl.ds(0, 2)])

        def jl_body(jl, c2):
            do_j(48 + jl, jl, it)
            return c2

        lax.fori_loop(0, 2, jl_body, 0)
        return carry

    lax.fori_loop(0, B_FULL_STEPS, full_step, 0)
    lax.fori_loop(0, B_TAIL_STEPS, tail_step, 0)


def kernel(x, table):
    tabT = table.T
    xT = x.astype(jnp.int32).T
    tail16 = table[N_VBLK * LANE:, :].reshape(16, LANE)
    t4 = _transpose_table(tabT, tail16)
    out_p = _gather_out(t4, xT)
    return jnp.transpose(out_p, (2, 0, 1))


# speedup vs baseline: 1.6324x; 1.3610x over previous
import functools

import jax
import jax.numpy as jnp
from jax import lax
from jax.experimental import pallas as pl
from jax.experimental.pallas import tpu as pltpu
from jax.experimental.pallas import tpu_sc as plsc

VOCAB = 1_000_000
DIM = 32
LANE = 128
NW = 32
N_VBLK = VOCAB // LANE
V_TAIL = VOCAB - N_VBLK * LANE
A_STEPS = (N_VBLK + NW - 1) // NW

NI_TILE = 16384 // LANE
N_FULL_UNIT = 6 * NI_TILE
B_FULL_STEPS = N_FULL_UNIT // NW
B_TAIL_STEPS = NI_TILE // NW

_mesh = plsc.VectorSubcoreMesh(core_axis_name="c", subcore_axis_name="s")
_params = pltpu.CompilerParams(needs_layout_passes=False)


def _iota16():
    return lax.iota(jnp.int32, 16)


@functools.partial(
    pl.kernel,
    mesh=_mesh,
    out_type=jax.ShapeDtypeStruct((VOCAB // 4, LANE), jnp.float32),
    scratch_types=[
        pltpu.VMEM((DIM, LANE), jnp.float32),
        pltpu.VMEM((DIM, LANE), jnp.float32),
    ],
    compiler_params=_params,
)
def _transpose_table(tabT_hbm, tail16_hbm, t4_hbm, in_v, out_v):
    wid = lax.axis_index("s") * 2 + lax.axis_index("c")

    lo16 = _iota16()
    hi16 = 16 + _iota16()

    def transpose_block():
        for q in range(32):
            for c in range(8):
                rows = lo16 if c % 2 == 0 else hi16
                cols = jnp.full((16,), q * 4 + c // 2, jnp.int32)
                out_v[q, pl.ds(c * 16, 16)] = plsc.load_gather(
                    in_v, [rows, cols])

    def step(k, carry):
        b = wid + k * NW

        @pl.when(b < N_VBLK)
        def _():
            pltpu.sync_copy(tabT_hbm.at[:, pl.ds(b * LANE, LANE)], in_v)
            transpose_block()
            pltpu.sync_copy(out_v, t4_hbm.at[pl.ds(b * 32, 32)])

        return carry

    lax.fori_loop(0, A_STEPS, step, 0)

    @pl.when(wid == 4)
    def _():
        pltpu.sync_copy(tail16_hbm, in_v.at[pl.ds(0, 16)])
        pltpu.sync_copy(
            in_v.at[pl.ds(0, 16)], t4_hbm.at[pl.ds(N_VBLK * 32, 16)])


@functools.partial(
    pl.kernel,
    mesh=_mesh,
    out_type=jax.ShapeDtypeStruct((50, DIM, 16384), jnp.float32),
    scratch_types=[
        pltpu.VMEM((8, LANE), jnp.int32),
        pltpu.VMEM((LANE,), jnp.int32),
        pltpu.VMEM((LANE,), jnp.int32),
        pltpu.VMEM((LANE, LANE), jnp.float32),
        pltpu.VMEM((DIM, LANE), jnp.float32),
        pltpu.SemaphoreType.DMA,
    ],
    compiler_params=_params,
)
def _gather_out(t4_hbm, xT_hbm, out_hbm, idx_v, tl_v, ml_v, buf_v, ob_v, sem):
    wid = lax.axis_index("s") * 2 + lax.axis_index("c")

    def do_j(jg, jl, it):
        for ci in range(8):
            v = idx_v[jl, pl.ds(ci * 16, 16)]
            tl_v[pl.ds(ci * 16, 16)] = lax.shift_right_logical(v, 2)
            ml_v[pl.ds(ci * 16, 16)] = (v & 3) * DIM
        pltpu.async_copy(t4_hbm.at[tl_v], buf_v, sem).wait()

        for ic in range(8):
            m = ml_v[pl.ds(ic * 16, 16)]
            rows = ic * 16 + _iota16()
            for f in range(DIM):
                ob_v[f, pl.ds(ic * 16, 16)] = plsc.load_gather(
                    buf_v, [rows, m + f])

        pltpu.sync_copy(ob_v, out_hbm.at[jg, :, pl.ds(it * LANE, LANE)])

    def full_step(k, carry):
        u = wid + k * NW
        jb = u // NI_TILE
        it = u - jb * NI_TILE
        pltpu.sync_copy(
            xT_hbm.at[pl.ds(jb * 8, 8), pl.ds(it * LANE, LANE)], idx_v)

        def jl_body(jl, c2):
            do_j(jb * 8 + jl, jl, it)
            return c2

        lax.fori_loop(0, 8, jl_body, 0)
        return carry

    def tail_step(k, carry):
        it = wid + k * NW
        pltpu.sync_copy(
            xT_hbm.at[pl.ds(48, 2), pl.ds(it * LANE, LANE)],
            idx_v.at[pl.ds(0, 2)])

        def jl_body(jl, c2):
            do_j(48 + jl, jl, it)
            return c2

        lax.fori_loop(0, 2, jl_body, 0)
        return carry

    lax.fori_loop(0, B_FULL_STEPS, full_step, 0)
    lax.fori_loop(0, B_TAIL_STEPS, tail_step, 0)


def kernel(x, table):
    xT = x.astype(jnp.int32).T
    t4 = jnp.ravel(table).reshape(VOCAB // 4, LANE)
    out_p = _gather_out(t4, xT)
    return jnp.transpose(out_p, (2, 0, 1))


# speedup vs baseline: 2.0012x; 1.2259x over previous
import functools

import jax
import jax.numpy as jnp
from jax import lax
from jax.experimental import pallas as pl
from jax.experimental.pallas import tpu as pltpu
from jax.experimental.pallas import tpu_sc as plsc

VOCAB = 1_000_000
DIM = 32
LANE = 128
NW = 32
N_VBLK = VOCAB // LANE
V_TAIL = VOCAB - N_VBLK * LANE
A_STEPS = (N_VBLK + NW - 1) // NW

NI_TILE = 16384 // LANE
N_FULL_UNIT = 6 * NI_TILE
B_FULL_STEPS = N_FULL_UNIT // NW
B_TAIL_STEPS = NI_TILE // NW

_mesh = plsc.VectorSubcoreMesh(core_axis_name="c", subcore_axis_name="s")
_params = pltpu.CompilerParams(needs_layout_passes=False)


def _iota16():
    return lax.iota(jnp.int32, 16)


@functools.partial(
    pl.kernel,
    mesh=_mesh,
    out_type=jax.ShapeDtypeStruct((VOCAB // 4, LANE), jnp.float32),
    scratch_types=[
        pltpu.VMEM((DIM, LANE), jnp.float32),
        pltpu.VMEM((DIM, LANE), jnp.float32),
    ],
    compiler_params=_params,
)
def _transpose_table(tabT_hbm, tail16_hbm, t4_hbm, in_v, out_v):
    wid = lax.axis_index("s") * 2 + lax.axis_index("c")

    lo16 = _iota16()
    hi16 = 16 + _iota16()

    def transpose_block():
        for q in range(32):
            for c in range(8):
                rows = lo16 if c % 2 == 0 else hi16
                cols = jnp.full((16,), q * 4 + c // 2, jnp.int32)
                out_v[q, pl.ds(c * 16, 16)] = plsc.load_gather(
                    in_v, [rows, cols])

    def step(k, carry):
        b = wid + k * NW

        @pl.when(b < N_VBLK)
        def _():
            pltpu.sync_copy(tabT_hbm.at[:, pl.ds(b * LANE, LANE)], in_v)
            transpose_block()
            pltpu.sync_copy(out_v, t4_hbm.at[pl.ds(b * 32, 32)])

        return carry

    lax.fori_loop(0, A_STEPS, step, 0)

    @pl.when(wid == 4)
    def _():
        pltpu.sync_copy(tail16_hbm, in_v.at[pl.ds(0, 16)])
        pltpu.sync_copy(
            in_v.at[pl.ds(0, 16)], t4_hbm.at[pl.ds(N_VBLK * 32, 16)])


@functools.partial(
    pl.kernel,
    mesh=_mesh,
    out_type=jax.ShapeDtypeStruct((50, DIM, 16384), jnp.float32),
    scratch_types=[
        pltpu.VMEM((8, LANE), jnp.int32),
        pltpu.VMEM((LANE,), jnp.int32),
        pltpu.VMEM((LANE,), jnp.int32),
        pltpu.VMEM((LANE,), jnp.int32),
        pltpu.VMEM((LANE,), jnp.int32),
        pltpu.VMEM((LANE, LANE), jnp.float32),
        pltpu.VMEM((LANE, LANE), jnp.float32),
        pltpu.VMEM((DIM, LANE), jnp.float32),
        pltpu.VMEM((DIM, LANE), jnp.float32),
        pltpu.SemaphoreType.DMA,
        pltpu.SemaphoreType.DMA,
    ],
    compiler_params=_params,
)
def _gather_out(t4_hbm, xT_hbm, out_hbm, idx_v, tlA, mlA, tlB, mlB,
                bufA, bufB, obA, obB, semG, semO):
    wid = lax.axis_index("s") * 2 + lax.axis_index("c")

    def build(jl, tl, ml):
        for ci in range(8):
            v = idx_v[jl, pl.ds(ci * 16, 16)]
            tl[pl.ds(ci * 16, 16)] = lax.shift_right_logical(v, 2)
            ml[pl.ds(ci * 16, 16)] = (v & 3) * DIM

    def fire(tl, buf):
        pltpu.async_copy(t4_hbm.at[tl], buf, semG)

    def gather_wait(buf):
        pltpu.make_async_copy(t4_hbm.at[tlA], buf, semG).wait()

    def transpose_into(buf, ob, ml):
        def ic_body(ic, c3):
            m = ml[pl.ds(ic * 16, 16)]
            rows = ic * 16 + _iota16()
            for f in range(DIM):
                ob[f, pl.ds(ic * 16, 16)] = plsc.load_gather(
                    buf, [rows, m + f])
            return c3

        lax.fori_loop(0, 8, ic_body, 0)

    def out_start(ob, jg, it):
        pltpu.async_copy(ob, out_hbm.at[jg, :, pl.ds(it * LANE, LANE)], semO)

    def out_wait():
        pltpu.make_async_copy(
            obA, out_hbm.at[0, :, pl.ds(0, LANE)], semO).wait()

    def full_step(k, carry):
        u = wid + k * NW
        jb = u // NI_TILE
        it = u - jb * NI_TILE
        jg0 = jb * 8
        pltpu.sync_copy(
            xT_hbm.at[pl.ds(jb * 8, 8), pl.ds(it * LANE, LANE)], idx_v)
        build(0, tlA, mlA)
        fire(tlA, bufA)

        def pair(g, c2):
            @pl.when(g > 0)
            def _():
                out_wait()
                out_wait()

            build(2 * g + 1, tlB, mlB)
            gather_wait(bufA)
            fire(tlB, bufB)
            transpose_into(bufA, obA, mlA)
            out_start(obA, jg0 + 2 * g, it)

            @pl.when(g < 3)
            def _():
                build(2 * g + 2, tlA, mlA)

            gather_wait(bufB)

            @pl.when(g < 3)
            def _():
                fire(tlA, bufA)

            transpose_into(bufB, obB, mlB)
            out_start(obB, jg0 + 2 * g + 1, it)
            return c2

        lax.fori_loop(0, 4, pair, 0)
        out_wait()
        out_wait()
        return carry

    def tail_step(k, carry):
        it = wid + k * NW
        pltpu.sync_copy(
            xT_hbm.at[pl.ds(48, 2), pl.ds(it * LANE, LANE)],
            idx_v.at[pl.ds(0, 2)])
        build(0, tlA, mlA)
        fire(tlA, bufA)
        build(1, tlB, mlB)
        gather_wait(bufA)
        fire(tlB, bufB)
        transpose_into(bufA, obA, mlA)
        out_start(obA, 48, it)
        gather_wait(bufB)
        transpose_into(bufB, obB, mlB)
        out_start(obB, 49, it)
        out_wait()
        out_wait()
        return carry

    lax.fori_loop(0, B_FULL_STEPS, full_step, 0)
    lax.fori_loop(0, B_TAIL_STEPS, tail_step, 0)


def kernel(x, table):
    xT = x.astype(jnp.int32).T
    t4 = jnp.ravel(table).reshape(VOCAB // 4, LANE)
    out_p = _gather_out(t4, xT)
    return jnp.transpose(out_p, (2, 0, 1))
